# baseline (speedup 1.0000x reference)
import functools

import jax
import jax.numpy as jnp
from jax import lax
from jax.experimental import pallas as pl
from jax.experimental.pallas import tpu as pltpu
from jax.experimental.pallas import tpu_sc as plsc

N = 10000
D = 128
H1 = 256
H2 = 128
RW = 5
DIN = D + RW

BM_HOP = 200
BM_MLP = 1000


def _hop_mm_kernel(hop_ref, y_ref, z_ref, s1_ref, s2_ref):
    h = hop_ref[...].astype(jnp.bfloat16)
    z = jax.lax.dot_general(h, y_ref[...], (((1,), (0,)), ((), ())),
                            preferred_element_type=jnp.float32)
    z_ref[...] = z

    @pl.when(pl.program_id(0) == 0)
    def _():
        s1_ref[...] = jnp.zeros_like(s1_ref)
        s2_ref[...] = jnp.zeros_like(s2_ref)

    s1_ref[...] += jnp.sum(z, axis=0, keepdims=True)
    s2_ref[...] += jnp.sum(z * z, axis=0, keepdims=True)


def _hop_mm(hop, y_bf16):
    return pl.pallas_call(
        _hop_mm_kernel,
        grid=(N // BM_HOP,),
        in_specs=[pl.BlockSpec((BM_HOP, N), lambda i: (i, 0)),
                  pl.BlockSpec((N, H1), lambda i: (0, 0))],
        out_specs=[pl.BlockSpec((BM_HOP, H1), lambda i: (i, 0)),
                   pl.BlockSpec((1, H1), lambda i: (0, 0)),
                   pl.BlockSpec((1, H1), lambda i: (0, 0))],
        out_shape=[jax.ShapeDtypeStruct((N, H1), jnp.float32),
                   jax.ShapeDtypeStruct((1, H1), jnp.float32),
                   jax.ShapeDtypeStruct((1, H1), jnp.float32)],
    )(hop, y_bf16)


def _yprep_kernel(xc_ref, w_ref, y_ref, s1_ref, s2_ref):
    y = jax.lax.dot_general(xc_ref[...], w_ref[0], (((1,), (0,)), ((), ())),
                            preferred_element_type=jnp.float32)
    y_ref[0] = y.astype(jnp.bfloat16)

    @pl.when(pl.program_id(1) == 0)
    def _():
        s1_ref[...] = jnp.zeros_like(s1_ref)
        s2_ref[...] = jnp.zeros_like(s2_ref)

    s1_ref[...] += jnp.sum(y, axis=0, keepdims=True)[None]
    s2_ref[...] += jnp.sum(y * y, axis=0, keepdims=True)[None]


def _yprep(xc_bf16, w1s_bf16):
    return pl.pallas_call(
        _yprep_kernel,
        grid=(4, N // BM_MLP),
        in_specs=[pl.BlockSpec((BM_MLP, DIN), lambda b, i: (i, 0)),
                  pl.BlockSpec((1, DIN, H1), lambda b, i: (b, 0, 0))],
        out_specs=[pl.BlockSpec((1, BM_MLP, H1), lambda b, i: (b, i, 0)),
                   pl.BlockSpec((1, 1, H1), lambda b, i: (b, 0, 0)),
                   pl.BlockSpec((1, 1, H1), lambda b, i: (b, 0, 0))],
        out_shape=[jax.ShapeDtypeStruct((4, N, H1), jnp.bfloat16),
                   jax.ShapeDtypeStruct((4, 1, H1), jnp.float32),
                   jax.ShapeDtypeStruct((4, 1, H1), jnp.float32)],
    )(xc_bf16, w1s_bf16)


def _passb_kernel(a_ref, m_ref, inv_ref, g_ref, be_ref, w2_ref,
                  b_ref, s1_ref, s2_ref):
    a = a_ref[...].astype(jnp.float32)
    h = (a - m_ref[...]) * inv_ref[...] * g_ref[...] + be_ref[...]
    h = jnp.maximum(h, 0.0).astype(jnp.bfloat16)
    b = jax.lax.dot_general(h, w2_ref[...], (((1,), (0,)), ((), ())),
                            preferred_element_type=jnp.float32)
    b_ref[...] = b

    @pl.when(pl.program_id(0) == 0)
    def _():
        s1_ref[...] = jnp.zeros_like(s1_ref)
        s2_ref[...] = jnp.zeros_like(s2_ref)

    s1_ref[...] += jnp.sum(b, axis=0, keepdims=True)
    s2_ref[...] += jnp.sum(b * b, axis=0, keepdims=True)


def _passb(a, m1, inv1, g1, be1, w2_bf16):
    return pl.pallas_call(
        _passb_kernel,
        grid=(N // BM_MLP,),
        in_specs=[pl.BlockSpec((BM_MLP, H1), lambda i: (i, 0)),
                  pl.BlockSpec((1, H1), lambda i: (0, 0)),
                  pl.BlockSpec((1, H1), lambda i: (0, 0)),
                  pl.BlockSpec((1, H1), lambda i: (0, 0)),
                  pl.BlockSpec((1, H1), lambda i: (0, 0)),
                  pl.BlockSpec((H1, H2), lambda i: (0, 0))],
        out_specs=[pl.BlockSpec((BM_MLP, H2), lambda i: (i, 0)),
                   pl.BlockSpec((1, H2), lambda i: (0, 0)),
                   pl.BlockSpec((1, H2), lambda i: (0, 0))],
        out_shape=[jax.ShapeDtypeStruct((N, H2), jnp.float32),
                   jax.ShapeDtypeStruct((1, H2), jnp.float32),
                   jax.ShapeDtypeStruct((1, H2), jnp.float32)],
    )(a, m1, inv1, g1, be1, w2_bf16)


def _passc_kernel(b0, b1, b2, b3, m_ref, inv_ref, g_ref, be_ref, w3_ref,
                  bias_ref, s_ref):
    acc = bias_ref[...] * jnp.ones((b0.shape[0], 1), jnp.float32)
    for i, bref in enumerate((b0, b1, b2, b3)):
        h = (bref[...] - m_ref[i]) * inv_ref[i] * g_ref[i] + be_ref[i]
        h = jnp.maximum(h, 0.0).astype(jnp.bfloat16)
        acc += jax.lax.dot_general(h, w3_ref[i], (((1,), (0,)), ((), ())),
                                   preferred_element_type=jnp.float32)
    s_ref[...] = acc


def _passc(bs, m2s, inv2s, g2s, be2s, w3s_bf16, bias_sum):
    vec = lambda: pl.BlockSpec((4, 1, H2), lambda i: (0, 0, 0))
    return pl.pallas_call(
        _passc_kernel,
        grid=(N // BM_MLP,),
        in_specs=[pl.BlockSpec((BM_MLP, H2), lambda i: (i, 0))] * 4
                 + [vec(), vec(), vec(), vec(),
                    pl.BlockSpec((4, H2, H2), lambda i: (0, 0, 0)),
                    pl.BlockSpec((1, H2), lambda i: (0, 0))],
        out_specs=pl.BlockSpec((BM_MLP, H2), lambda i: (i, 0)),
        out_shape=jax.ShapeDtypeStruct((N, H2), jnp.float32),
    )(*bs, m2s, inv2s, g2s, be2s, w3s_bf16, bias_sum)


def _finalize_stats(s1, s2):
    m = s1 / N
    v = jnp.maximum(s2 / N - m * m, 0.0)
    return m, jax.lax.rsqrt(v + 1e-5)


N_PAD = 10240
STRIPE = N_PAD // 16
K_E = 256
E_TOT = 320000 + N
PER_TILE = 10752
NB = 1
Q = PER_TILE // NB
CQ = Q // K_E
E_PAD = 32 * PER_TILE
K_DEG = 256
_SC_MESH = dict(core_axis_name="c", subcore_axis_name="s")


def _deg_sc(dst_pad):
    @functools.partial(
        pl.kernel,
        out_type=jax.ShapeDtypeStruct((2, N_PAD, 16), jnp.float32),
        mesh=plsc.VectorSubcoreMesh(**_SC_MESH),
        compiler_params=pltpu.CompilerParams(use_tc_tiling_on_sc=False),
        scratch_types=[
            pltpu.VMEM((K_DEG,), jnp.int32),
            pltpu.VMEM((K_DEG, 16), jnp.float32),
            pltpu.VMEM_SHARED((N_PAD, 16), jnp.float32),
            pltpu.VMEM((STRIPE, 16), jnp.float32),
        ],
    )
    def k(dst_hbm, out, dbuf, ones_v, acc, zbuf):
        cid = lax.axis_index("c")
        sid = lax.axis_index("s")
        wid = cid * 16 + sid
        def fill(i, carry):
            ones_v[i, pl.ds(0, 16)] = jnp.ones((16,), jnp.float32)
            return carry
        lax.fori_loop(0, K_DEG, fill, 0)
        def zf(i, carry):
            zbuf[i, pl.ds(0, 16)] = jnp.zeros((16,), jnp.float32)
            return carry
        lax.fori_loop(0, STRIPE, zf, 0)
        pltpu.sync_copy(zbuf, acc.at[pl.ds(sid * STRIPE, STRIPE)])
        plsc.subcore_barrier()

        def chunk(c, carry):
            pltpu.sync_copy(
                dst_hbm.at[pl.ds(wid * PER_TILE + c * K_DEG, K_DEG)], dbuf)
            pltpu.sync_copy(ones_v, acc.at[dbuf], add=True)
            return carry
        lax.fori_loop(0, PER_TILE // K_DEG, chunk, 0)
        plsc.subcore_barrier()
        pltpu.sync_copy(acc.at[pl.ds(sid * STRIPE, STRIPE)],
                        out.at[cid, pl.ds(sid * STRIPE, STRIPE)])

    return k(dst_pad)


def _gcn_agg_sc(src_pad, dst_pad, tabs, zer):
    nt = len(tabs)

    @functools.partial(
        pl.kernel,
        out_type=jax.ShapeDtypeStruct((2, nt, N_PAD, 128), jnp.bfloat16),
        mesh=plsc.VectorSubcoreMesh(**_SC_MESH),
        compiler_params=pltpu.CompilerParams(use_tc_tiling_on_sc=False),
        scratch_types=[
            pltpu.VMEM((Q,), jnp.int32),
            pltpu.VMEM((Q,), jnp.int32),
            pltpu.VMEM((2 * K_E, 128), jnp.bfloat16),
            pltpu.VMEM_SHARED((N_PAD, 128), jnp.bfloat16),
            pltpu.SemaphoreType.DMA,
            pltpu.SemaphoreType.DMA,
            pltpu.SemaphoreType.DMA,
            pltpu.SemaphoreType.DMA,
        ],
    )
    def k(src_hbm, dst_hbm, zer_hbm, *rest):
        tab_refs = rest[:nt]
        out = rest[nt]
        sidx, didx, rows, acc, sem0, sem1, sem2, sem3 = rest[nt + 1:]
        sems = (sem0, sem1)
        ssems = (sem2, sem3)
        cid = lax.axis_index("c")
        sid = lax.axis_index("s")
        wid = cid * 16 + sid
        ebase = wid * PER_TILE

        for t in range(nt):
            tab = tab_refs[t]
            pltpu.sync_copy(zer_hbm, acc.at[pl.ds(sid * STRIPE, STRIPE)])
            plsc.subcore_barrier()

            for n in range(NB):
                qbase = ebase + n * Q
                pltpu.sync_copy(src_hbm.at[pl.ds(qbase, Q)], sidx)
                pltpu.sync_copy(dst_hbm.at[pl.ds(qbase, Q)], didx)
                pltpu.async_copy(tab.at[sidx.at[pl.ds(0, K_E)]],
                                 rows.at[pl.ds(0, K_E)], sems[0])

                def pair(j, carry):
                    for b in (0, 1):
                        c = 2 * j + b
                        nxt = 1 - b
                        pltpu.make_async_copy(
                            tab.at[sidx.at[pl.ds(c * K_E, K_E)]],
                            rows.at[pl.ds(b * K_E, K_E)], sems[b]).wait()
                        pltpu.async_copy(rows.at[pl.ds(b * K_E, K_E)],
                                         acc.at[didx.at[pl.ds(c * K_E, K_E)]],
                                         ssems[b], add=True)
                        if b == 0:
                            @pl.when(j >= 1)
                            def _():
                                pltpu.make_async_copy(
                                    rows.at[pl.ds(K_E, K_E)],
                                    acc.at[didx.at[pl.ds((c - 1) * K_E, K_E)]],
                                    ssems[1]).wait()
                            pltpu.async_copy(
                                tab.at[sidx.at[pl.ds((c + 1) * K_E, K_E)]],
                                rows.at[pl.ds(K_E, K_E)], sems[1])
                        else:
                            pltpu.make_async_copy(
                                rows.at[pl.ds(0, K_E)],
                                acc.at[didx.at[pl.ds((c - 1) * K_E, K_E)]],
                                ssems[0]).wait()
                            @pl.when(c + 1 < CQ)
                            def _():
                                pltpu.async_copy(
                                    tab.at[sidx.at[pl.ds((c + 1) * K_E, K_E)]],
                                    rows.at[pl.ds(0, K_E)], sems[0])
                    return carry
                lax.fori_loop(0, CQ // 2, pair, 0)
                pltpu.make_async_copy(
                    rows.at[pl.ds(K_E, K_E)],
                    acc.at[didx.at[pl.ds((CQ - 1) * K_E, K_E)]],
                    ssems[1]).wait()

            plsc.subcore_barrier()
            pltpu.sync_copy(acc.at[pl.ds(sid * STRIPE, STRIPE)],
                            out.at[cid, t, pl.ds(sid * STRIPE, STRIPE)])
            plsc.subcore_barrier()

    return k(src_pad, dst_pad, zer, *tabs)


def _dis_block(degp):
    deg = degp[0, :, :1] + degp[1, :, :1]
    return jax.lax.rsqrt(jnp.maximum(deg, 1e-12))


def _psum(aggp_ref, c):
    return (aggp_ref[0, c].astype(jnp.float32)
            + aggp_ref[1, c].astype(jnp.float32))


def _prep1_kernel(x_ref, degp_ref, o_ref):
    o_ref[...] = (x_ref[...] * _dis_block(degp_ref)).astype(jnp.bfloat16)


def _prep2_kernel(aggp_ref, degp_ref, w1_ref, b1_ref, w2_ref, o_ref):
    dis = _dis_block(degp_ref)
    p = _psum(aggp_ref, 0) * dis
    h = jax.lax.dot_general(p, w1_ref[...], (((1,), (0,)), ((), ())),
                            preferred_element_type=jnp.float32) + b1_ref[...]
    h = jnp.maximum(h, 0.0)
    o_ref[...] = (jax.lax.dot_general(
        h, w2_ref[...], (((1,), (0,)), ((), ())),
        preferred_element_type=jnp.float32) * dis).astype(jnp.bfloat16)


def _prep23_kernel(nc, aggp_ref, degp_ref, w_ref, bias_ref, o_ref):
    dis = _dis_block(degp_ref)
    hs = []
    for c in range(nc):
        p = _psum(aggp_ref, c)
        hs.append(jnp.maximum(p * dis + bias_ref[:, c * 128:(c + 1) * 128],
                              0.0))
    h = jnp.concatenate(hs, axis=1) if nc > 1 else hs[0]
    o_ref[...] = (jax.lax.dot_general(
        h, w_ref[...], (((1,), (0,)), ((), ())),
        preferred_element_type=jnp.float32) * dis).astype(o_ref.dtype)


def _epi_kernel(aggp_ref, degp_ref, bias_ref, o_ref):
    dis = _dis_block(degp_ref)
    o_ref[...] = jnp.maximum(_psum(aggp_ref, 0) * dis + bias_ref[...], 0.0)


_BMG = 2000


def _gcn_tc_specs(nc, hw):
    return [pl.BlockSpec((2, nc, _BMG, 128), lambda i: (0, 0, i, 0)),
            pl.BlockSpec((2, _BMG, 16), lambda i: (0, i, 0)),
            pl.BlockSpec(hw, lambda i: (0, 0)),
            pl.BlockSpec((1, 128 * nc), lambda i: (0, 0))]


def _gcn(x, edges, params):
    loops = jnp.arange(N, dtype=jnp.int32)
    fill_s = jnp.full((E_PAD - E_TOT,), N, jnp.int32)
    fill_d = jnp.full((32 * PER_TILE - E_TOT,), N, jnp.int32)
    src = jnp.concatenate([edges[0], loops, fill_s])
    dst = jnp.concatenate([edges[1], loops, fill_d])

    degp = _deg_sc(dst)
    zer = jnp.zeros((STRIPE, 128), jnp.bfloat16)

    t1 = pl.pallas_call(
        _prep1_kernel,
        grid=(N // _BMG,),
        in_specs=[pl.BlockSpec((_BMG, D), lambda i: (i, 0)),
                  pl.BlockSpec((2, _BMG, 16), lambda i: (0, i, 0))],
        out_specs=pl.BlockSpec((_BMG, 128), lambda i: (i, 0)),
        out_shape=jax.ShapeDtypeStruct((N_PAD, 128), jnp.bfloat16),
    )(x, degp)
    agg1 = _gcn_agg_sc(src, dst, [t1], zer)

    xw2 = pl.pallas_call(
        _prep2_kernel,
        grid=(N // _BMG,),
        in_specs=[pl.BlockSpec((2, 1, _BMG, 128), lambda i: (0, 0, i, 0)),
                  pl.BlockSpec((2, _BMG, 16), lambda i: (0, i, 0)),
                  pl.BlockSpec((D, H1), lambda i: (0, 0)),
                  pl.BlockSpec((1, H1), lambda i: (0, 0)),
                  pl.BlockSpec((H1, H2), lambda i: (0, 0))],
        out_specs=pl.BlockSpec((_BMG, 128), lambda i: (i, 0)),
        out_shape=jax.ShapeDtypeStruct((N_PAD, 128), jnp.bfloat16),
    )(agg1, degp, params['gcn1']['W'], params['gcn1']['b'][None],
      params['gcn2']['W'])
    agg2 = _gcn_agg_sc(src, dst, [xw2], zer)

    xw3 = pl.pallas_call(
        functools.partial(_prep23_kernel, 1),
        grid=(N // _BMG,),
        in_specs=_gcn_tc_specs(1, (H2, H2)),
        out_specs=pl.BlockSpec((_BMG, 128), lambda i: (i, 0)),
        out_shape=jax.ShapeDtypeStruct((N_PAD, 128), jnp.bfloat16),
    )(agg2, degp, params['gcn3']['W'], params['gcn2']['b'][None])
    agg3 = _gcn_agg_sc(src, dst, [xw3], zer)

    return pl.pallas_call(
        _epi_kernel,
        grid=(N // _BMG,),
        in_specs=_gcn_tc_specs(1, (H2, H2))[:2]
                 + [pl.BlockSpec((1, H2), lambda i: (0, 0))],
        out_specs=pl.BlockSpec((_BMG, H2), lambda i: (i, 0)),
        out_shape=jax.ShapeDtypeStruct((N, H2), jnp.float32),
    )(agg3, degp, params['gcn3']['b'][None])


def kernel(x, edges, walk_feats, hop1, hop2, hop3, params):
    xc = jnp.concatenate([x, walk_feats[:, :RW]], axis=1).astype(jnp.bfloat16)
    w1s = jnp.stack([params['sub%d' % i]['W1'] for i in range(4)]
                    ).astype(jnp.bfloat16)
    ys, ys1, ys2 = _yprep(xc, w1s)

    zs, zs1, zs2 = [None] * 4, [ys1[0]] * 4, [ys2[0]] * 4
    zs[0] = ys[0].astype(jnp.float32)
    for i, hop in enumerate((hop1, hop2, hop3)):
        zs[i + 1], zs1[i + 1], zs2[i + 1] = _hop_mm(hop, ys[i + 1])

    bs, bs1, bs2 = [None] * 4, [None] * 4, [None] * 4
    for i in range(4):
        p = params['sub%d' % i]
        m1, inv1 = _finalize_stats(zs1[i], zs2[i])
        bs[i], bs1[i], bs2[i] = _passb(
            zs[i], m1, inv1, p['g1'][None], p['be1'][None],
            p['W2'].astype(jnp.bfloat16))

    m2s, inv2s = [], []
    for i in range(4):
        m2, inv2 = _finalize_stats(bs1[i], bs2[i])
        m2s.append(m2)
        inv2s.append(inv2)
    m2s = jnp.stack(m2s)
    inv2s = jnp.stack(inv2s)
    g2s = jnp.stack([params['sub%d' % i]['g2'][None] for i in range(4)])
    be2s = jnp.stack([params['sub%d' % i]['be2'][None] for i in range(4)])
    w3s = jnp.stack([params['sub%d' % i]['W3'] for i in range(4)]
                    ).astype(jnp.bfloat16)
    bias_sum = sum(params['sub%d' % i]['b3'] for i in range(4))[None]
    s = _passc(bs, m2s, inv2s, g2s, be2s, w3s, bias_sum)

    h = _gcn(x, edges, params)
    return jnp.concatenate([h, s], axis=1)


# speedup vs baseline: 2.0247x; 2.0247x over previous
import functools

import jax
import jax.numpy as jnp
from jax import lax
from jax.experimental import pallas as pl
from jax.experimental.pallas import tpu as pltpu
from jax.experimental.pallas import tpu_sc as plsc

N = 10000
D = 128
H1 = 256
H2 = 128
RW = 5
DIN = D + RW

BM_HOP = 200
BM_MLP = 1000


def _hop_mm_kernel(hop_ref, y_ref, z_ref, s1_ref, s2_ref):
    h = hop_ref[...].astype(jnp.bfloat16)
    z = jax.lax.dot_general(h, y_ref[...], (((1,), (0,)), ((), ())),
                            preferred_element_type=jnp.float32)
    z_ref[...] = z

    @pl.when(pl.program_id(0) == 0)
    def _():
        s1_ref[...] = jnp.zeros_like(s1_ref)
        s2_ref[...] = jnp.zeros_like(s2_ref)

    s1_ref[...] += jnp.sum(z, axis=0, keepdims=True)
    s2_ref[...] += jnp.sum(z * z, axis=0, keepdims=True)


def _hop_mm(hop, y_bf16):
    return pl.pallas_call(
        _hop_mm_kernel,
        grid=(N // BM_HOP,),
        in_specs=[pl.BlockSpec((BM_HOP, N), lambda i: (i, 0)),
                  pl.BlockSpec((N, H1), lambda i: (0, 0))],
        out_specs=[pl.BlockSpec((BM_HOP, H1), lambda i: (i, 0)),
                   pl.BlockSpec((1, H1), lambda i: (0, 0)),
                   pl.BlockSpec((1, H1), lambda i: (0, 0))],
        out_shape=[jax.ShapeDtypeStruct((N, H1), jnp.float32),
                   jax.ShapeDtypeStruct((1, H1), jnp.float32),
                   jax.ShapeDtypeStruct((1, H1), jnp.float32)],
    )(hop, y_bf16)


def _yprep_kernel(xc_ref, w_ref, y_ref, s1_ref, s2_ref):
    y = jax.lax.dot_general(xc_ref[...], w_ref[0], (((1,), (0,)), ((), ())),
                            preferred_element_type=jnp.float32)
    y_ref[0] = y.astype(jnp.bfloat16)

    @pl.when(pl.program_id(1) == 0)
    def _():
        s1_ref[...] = jnp.zeros_like(s1_ref)
        s2_ref[...] = jnp.zeros_like(s2_ref)

    s1_ref[...] += jnp.sum(y, axis=0, keepdims=True)[None]
    s2_ref[...] += jnp.sum(y * y, axis=0, keepdims=True)[None]


def _yprep(xc_bf16, w1s_bf16):
    return pl.pallas_call(
        _yprep_kernel,
        grid=(4, N // BM_MLP),
        in_specs=[pl.BlockSpec((BM_MLP, DIN), lambda b, i: (i, 0)),
                  pl.BlockSpec((1, DIN, H1), lambda b, i: (b, 0, 0))],
        out_specs=[pl.BlockSpec((1, BM_MLP, H1), lambda b, i: (b, i, 0)),
                   pl.BlockSpec((1, 1, H1), lambda b, i: (b, 0, 0)),
                   pl.BlockSpec((1, 1, H1), lambda b, i: (b, 0, 0))],
        out_shape=[jax.ShapeDtypeStruct((4, N, H1), jnp.bfloat16),
                   jax.ShapeDtypeStruct((4, 1, H1), jnp.float32),
                   jax.ShapeDtypeStruct((4, 1, H1), jnp.float32)],
    )(xc_bf16, w1s_bf16)


def _passb_kernel(a_ref, m_ref, inv_ref, g_ref, be_ref, w2_ref,
                  b_ref, s1_ref, s2_ref):
    a = a_ref[...].astype(jnp.float32)
    h = (a - m_ref[...]) * inv_ref[...] * g_ref[...] + be_ref[...]
    h = jnp.maximum(h, 0.0).astype(jnp.bfloat16)
    b = jax.lax.dot_general(h, w2_ref[...], (((1,), (0,)), ((), ())),
                            preferred_element_type=jnp.float32)
    b_ref[...] = b

    @pl.when(pl.program_id(0) == 0)
    def _():
        s1_ref[...] = jnp.zeros_like(s1_ref)
        s2_ref[...] = jnp.zeros_like(s2_ref)

    s1_ref[...] += jnp.sum(b, axis=0, keepdims=True)
    s2_ref[...] += jnp.sum(b * b, axis=0, keepdims=True)


def _passb(a, m1, inv1, g1, be1, w2_bf16):
    return pl.pallas_call(
        _passb_kernel,
        grid=(N // BM_MLP,),
        in_specs=[pl.BlockSpec((BM_MLP, H1), lambda i: (i, 0)),
                  pl.BlockSpec((1, H1), lambda i: (0, 0)),
                  pl.BlockSpec((1, H1), lambda i: (0, 0)),
                  pl.BlockSpec((1, H1), lambda i: (0, 0)),
                  pl.BlockSpec((1, H1), lambda i: (0, 0)),
                  pl.BlockSpec((H1, H2), lambda i: (0, 0))],
        out_specs=[pl.BlockSpec((BM_MLP, H2), lambda i: (i, 0)),
                   pl.BlockSpec((1, H2), lambda i: (0, 0)),
                   pl.BlockSpec((1, H2), lambda i: (0, 0))],
        out_shape=[jax.ShapeDtypeStruct((N, H2), jnp.float32),
                   jax.ShapeDtypeStruct((1, H2), jnp.float32),
                   jax.ShapeDtypeStruct((1, H2), jnp.float32)],
    )(a, m1, inv1, g1, be1, w2_bf16)


def _passc_kernel(b0, b1, b2, b3, m_ref, inv_ref, g_ref, be_ref, w3_ref,
                  bias_ref, s_ref):
    acc = bias_ref[...] * jnp.ones((b0.shape[0], 1), jnp.float32)
    for i, bref in enumerate((b0, b1, b2, b3)):
        h = (bref[...] - m_ref[i]) * inv_ref[i] * g_ref[i] + be_ref[i]
        h = jnp.maximum(h, 0.0).astype(jnp.bfloat16)
        acc += jax.lax.dot_general(h, w3_ref[i], (((1,), (0,)), ((), ())),
                                   preferred_element_type=jnp.float32)
    s_ref[...] = acc


def _passc(bs, m2s, inv2s, g2s, be2s, w3s_bf16, bias_sum):
    vec = lambda: pl.BlockSpec((4, 1, H2), lambda i: (0, 0, 0))
    return pl.pallas_call(
        _passc_kernel,
        grid=(N // BM_MLP,),
        in_specs=[pl.BlockSpec((BM_MLP, H2), lambda i: (i, 0))] * 4
                 + [vec(), vec(), vec(), vec(),
                    pl.BlockSpec((4, H2, H2), lambda i: (0, 0, 0)),
                    pl.BlockSpec((1, H2), lambda i: (0, 0))],
        out_specs=pl.BlockSpec((BM_MLP, H2), lambda i: (i, 0)),
        out_shape=jax.ShapeDtypeStruct((N, H2), jnp.float32),
    )(*bs, m2s, inv2s, g2s, be2s, w3s_bf16, bias_sum)


def _finalize_stats(s1, s2):
    m = s1 / N
    v = jnp.maximum(s2 / N - m * m, 0.0)
    return m, jax.lax.rsqrt(v + 1e-5)


N_PAD = 10240
STRIPE = N_PAD // 16
K_E = 128
E_TOT = 320000 + N
PER_TILE = 10752
NB = 1
Q = PER_TILE // NB
CQ = Q // K_E
E_PAD = 32 * PER_TILE
K_DEG = 256
_SC_MESH = dict(core_axis_name="c", subcore_axis_name="s")


def _deg_sc(dst_pad):
    @functools.partial(
        pl.kernel,
        out_type=jax.ShapeDtypeStruct((2, N_PAD, 16), jnp.float32),
        mesh=plsc.VectorSubcoreMesh(**_SC_MESH),
        compiler_params=pltpu.CompilerParams(use_tc_tiling_on_sc=False),
        scratch_types=[
            pltpu.VMEM((K_DEG,), jnp.int32),
            pltpu.VMEM((K_DEG, 16), jnp.float32),
            pltpu.VMEM_SHARED((N_PAD, 16), jnp.float32),
            pltpu.VMEM((STRIPE, 16), jnp.float32),
        ],
    )
    def k(dst_hbm, out, dbuf, ones_v, acc, zbuf):
        cid = lax.axis_index("c")
        sid = lax.axis_index("s")
        wid = cid * 16 + sid
        def fill(i, carry):
            ones_v[i, pl.ds(0, 16)] = jnp.ones((16,), jnp.float32)
            return carry
        lax.fori_loop(0, K_DEG, fill, 0)
        def zf(i, carry):
            zbuf[i, pl.ds(0, 16)] = jnp.zeros((16,), jnp.float32)
            return carry
        lax.fori_loop(0, STRIPE, zf, 0)
        pltpu.sync_copy(zbuf, acc.at[pl.ds(sid * STRIPE, STRIPE)])
        plsc.subcore_barrier()

        def chunk(c, carry):
            pltpu.sync_copy(
                dst_hbm.at[pl.ds(wid * PER_TILE + c * K_DEG, K_DEG)], dbuf)
            pltpu.sync_copy(ones_v, acc.at[dbuf], add=True)
            return carry
        lax.fori_loop(0, PER_TILE // K_DEG, chunk, 0)
        plsc.subcore_barrier()
        pltpu.sync_copy(acc.at[pl.ds(sid * STRIPE, STRIPE)],
                        out.at[cid, pl.ds(sid * STRIPE, STRIPE)])

    return k(dst_pad)


def _gcn_agg_sc(src_pad, dst_pad, tabs, zer):
    nt = len(tabs)

    @functools.partial(
        pl.kernel,
        out_type=jax.ShapeDtypeStruct((2, nt, N_PAD, 128), jnp.bfloat16),
        mesh=plsc.VectorSubcoreMesh(**_SC_MESH),
        compiler_params=pltpu.CompilerParams(use_tc_tiling_on_sc=False),
        scratch_types=[
            pltpu.VMEM((Q,), jnp.int32),
            pltpu.VMEM((Q,), jnp.int32),
            pltpu.VMEM((2 * K_E, 128), jnp.bfloat16),
            pltpu.VMEM_SHARED((N_PAD, 128), jnp.bfloat16),
            pltpu.VMEM_SHARED((N_PAD, 128), jnp.bfloat16),
            pltpu.SemaphoreType.DMA,
            pltpu.SemaphoreType.DMA,
            pltpu.SemaphoreType.DMA,
            pltpu.SemaphoreType.DMA,
        ],
    )
    def k(src_hbm, dst_hbm, zer_hbm, *rest):
        tab_refs = rest[:nt]
        out = rest[nt]
        sidx, didx, rows, acc, tsp, sem0, sem1, sem2, sem3 = rest[nt + 1:]
        sems = (sem0, sem1)
        ssems = (sem2, sem3)
        cid = lax.axis_index("c")
        sid = lax.axis_index("s")
        wid = cid * 16 + sid
        ebase = wid * PER_TILE

        for t in range(nt):
            pltpu.sync_copy(tab_refs[t].at[pl.ds(sid * STRIPE, STRIPE)],
                            tsp.at[pl.ds(sid * STRIPE, STRIPE)])
            pltpu.sync_copy(zer_hbm, acc.at[pl.ds(sid * STRIPE, STRIPE)])
            plsc.subcore_barrier()
            tab = tsp

            for n in range(NB):
                qbase = ebase + n * Q
                pltpu.sync_copy(src_hbm.at[pl.ds(qbase, Q)], sidx)
                pltpu.sync_copy(dst_hbm.at[pl.ds(qbase, Q)], didx)
                pltpu.async_copy(tab.at[sidx.at[pl.ds(0, K_E)]],
                                 rows.at[pl.ds(0, K_E)], sems[0])

                def pair(j, carry):
                    for b in (0, 1):
                        c = 2 * j + b
                        nxt = 1 - b
                        pltpu.make_async_copy(
                            tab.at[sidx.at[pl.ds(c * K_E, K_E)]],
                            rows.at[pl.ds(b * K_E, K_E)], sems[b]).wait()
                        pltpu.async_copy(rows.at[pl.ds(b * K_E, K_E)],
                                         acc.at[didx.at[pl.ds(c * K_E, K_E)]],
                                         ssems[b], add=True)
                        if b == 0:
                            @pl.when(j >= 1)
                            def _():
                                pltpu.make_async_copy(
                                    rows.at[pl.ds(K_E, K_E)],
                                    acc.at[didx.at[pl.ds((c - 1) * K_E, K_E)]],
                                    ssems[1]).wait()
                            pltpu.async_copy(
                                tab.at[sidx.at[pl.ds((c + 1) * K_E, K_E)]],
                                rows.at[pl.ds(K_E, K_E)], sems[1])
                        else:
                            pltpu.make_async_copy(
                                rows.at[pl.ds(0, K_E)],
                                acc.at[didx.at[pl.ds((c - 1) * K_E, K_E)]],
                                ssems[0]).wait()
                            @pl.when(c + 1 < CQ)
                            def _():
                                pltpu.async_copy(
                                    tab.at[sidx.at[pl.ds((c + 1) * K_E, K_E)]],
                                    rows.at[pl.ds(0, K_E)], sems[0])
                    return carry
                lax.fori_loop(0, CQ // 2, pair, 0)
                pltpu.make_async_copy(
                    rows.at[pl.ds(K_E, K_E)],
                    acc.at[didx.at[pl.ds((CQ - 1) * K_E, K_E)]],
                    ssems[1]).wait()

            plsc.subcore_barrier()
            pltpu.sync_copy(acc.at[pl.ds(sid * STRIPE, STRIPE)],
                            out.at[cid, t, pl.ds(sid * STRIPE, STRIPE)])
            plsc.subcore_barrier()

    return k(src_pad, dst_pad, zer, *tabs)


def _dis_block(degp):
    deg = degp[0, :, :1] + degp[1, :, :1]
    return jax.lax.rsqrt(jnp.maximum(deg, 1e-12))


def _psum(aggp_ref, c):
    return (aggp_ref[0, c].astype(jnp.float32)
            + aggp_ref[1, c].astype(jnp.float32))


def _prep1_kernel(x_ref, degp_ref, o_ref):
    o_ref[...] = (x_ref[...] * _dis_block(degp_ref)).astype(jnp.bfloat16)


def _prep2_kernel(aggp_ref, degp_ref, w1_ref, b1_ref, w2_ref, o_ref):
    dis = _dis_block(degp_ref)
    p = _psum(aggp_ref, 0) * dis
    h = jax.lax.dot_general(p, w1_ref[...], (((1,), (0,)), ((), ())),
                            preferred_element_type=jnp.float32) + b1_ref[...]
    h = jnp.maximum(h, 0.0)
    o_ref[...] = (jax.lax.dot_general(
        h, w2_ref[...], (((1,), (0,)), ((), ())),
        preferred_element_type=jnp.float32) * dis).astype(jnp.bfloat16)


def _prep23_kernel(nc, aggp_ref, degp_ref, w_ref, bias_ref, o_ref):
    dis = _dis_block(degp_ref)
    hs = []
    for c in range(nc):
        p = _psum(aggp_ref, c)
        hs.append(jnp.maximum(p * dis + bias_ref[:, c * 128:(c + 1) * 128],
                              0.0))
    h = jnp.concatenate(hs, axis=1) if nc > 1 else hs[0]
    o_ref[...] = (jax.lax.dot_general(
        h, w_ref[...], (((1,), (0,)), ((), ())),
        preferred_element_type=jnp.float32) * dis).astype(o_ref.dtype)


def _epi_kernel(aggp_ref, degp_ref, bias_ref, o_ref):
    dis = _dis_block(degp_ref)
    o_ref[...] = jnp.maximum(_psum(aggp_ref, 0) * dis + bias_ref[...], 0.0)


_BMG = 2000


def _gcn_tc_specs(nc, hw):
    return [pl.BlockSpec((2, nc, _BMG, 128), lambda i: (0, 0, i, 0)),
            pl.BlockSpec((2, _BMG, 16), lambda i: (0, i, 0)),
            pl.BlockSpec(hw, lambda i: (0, 0)),
            pl.BlockSpec((1, 128 * nc), lambda i: (0, 0))]


def _gcn(x, edges, params):
    loops = jnp.arange(N, dtype=jnp.int32)
    fill_s = jnp.full((E_PAD - E_TOT,), N, jnp.int32)
    fill_d = jnp.full((32 * PER_TILE - E_TOT,), N, jnp.int32)
    src = jnp.concatenate([edges[0], loops, fill_s])
    dst = jnp.concatenate([edges[1], loops, fill_d])

    degp = _deg_sc(dst)
    zer = jnp.zeros((STRIPE, 128), jnp.bfloat16)

    t1 = pl.pallas_call(
        _prep1_kernel,
        grid=(N // _BMG,),
        in_specs=[pl.BlockSpec((_BMG, D), lambda i: (i, 0)),
                  pl.BlockSpec((2, _BMG, 16), lambda i: (0, i, 0))],
        out_specs=pl.BlockSpec((_BMG, 128), lambda i: (i, 0)),
        out_shape=jax.ShapeDtypeStruct((N_PAD, 128), jnp.bfloat16),
    )(x, degp)
    agg1 = _gcn_agg_sc(src, dst, [t1], zer)

    xw2 = pl.pallas_call(
        _prep2_kernel,
        grid=(N // _BMG,),
        in_specs=[pl.BlockSpec((2, 1, _BMG, 128), lambda i: (0, 0, i, 0)),
                  pl.BlockSpec((2, _BMG, 16), lambda i: (0, i, 0)),
                  pl.BlockSpec((D, H1), lambda i: (0, 0)),
                  pl.BlockSpec((1, H1), lambda i: (0, 0)),
                  pl.BlockSpec((H1, H2), lambda i: (0, 0))],
        out_specs=pl.BlockSpec((_BMG, 128), lambda i: (i, 0)),
        out_shape=jax.ShapeDtypeStruct((N_PAD, 128), jnp.bfloat16),
    )(agg1, degp, params['gcn1']['W'], params['gcn1']['b'][None],
      params['gcn2']['W'])
    agg2 = _gcn_agg_sc(src, dst, [xw2], zer)

    xw3 = pl.pallas_call(
        functools.partial(_prep23_kernel, 1),
        grid=(N // _BMG,),
        in_specs=_gcn_tc_specs(1, (H2, H2)),
        out_specs=pl.BlockSpec((_BMG, 128), lambda i: (i, 0)),
        out_shape=jax.ShapeDtypeStruct((N_PAD, 128), jnp.bfloat16),
    )(agg2, degp, params['gcn3']['W'], params['gcn2']['b'][None])
    agg3 = _gcn_agg_sc(src, dst, [xw3], zer)

    return pl.pallas_call(
        _epi_kernel,
        grid=(N // _BMG,),
        in_specs=_gcn_tc_specs(1, (H2, H2))[:2]
                 + [pl.BlockSpec((1, H2), lambda i: (0, 0))],
        out_specs=pl.BlockSpec((_BMG, H2), lambda i: (i, 0)),
        out_shape=jax.ShapeDtypeStruct((N, H2), jnp.float32),
    )(agg3, degp, params['gcn3']['b'][None])


def kernel(x, edges, walk_feats, hop1, hop2, hop3, params):
    xc = jnp.concatenate([x, walk_feats[:, :RW]], axis=1).astype(jnp.bfloat16)
    w1s = jnp.stack([params['sub%d' % i]['W1'] for i in range(4)]
                    ).astype(jnp.bfloat16)
    ys, ys1, ys2 = _yprep(xc, w1s)

    zs, zs1, zs2 = [None] * 4, [ys1[0]] * 4, [ys2[0]] * 4
    zs[0] = ys[0].astype(jnp.float32)
    for i, hop in enumerate((hop1, hop2, hop3)):
        zs[i + 1], zs1[i + 1], zs2[i + 1] = _hop_mm(hop, ys[i + 1])

    bs, bs1, bs2 = [None] * 4, [None] * 4, [None] * 4
    for i in range(4):
        p = params['sub%d' % i]
        m1, inv1 = _finalize_stats(zs1[i], zs2[i])
        bs[i], bs1[i], bs2[i] = _passb(
            zs[i], m1, inv1, p['g1'][None], p['be1'][None],
            p['W2'].astype(jnp.bfloat16))

    m2s, inv2s = [], []
    for i in range(4):
        m2, inv2 = _finalize_stats(bs1[i], bs2[i])
        m2s.append(m2)
        inv2s.append(inv2)
    m2s = jnp.stack(m2s)
    inv2s = jnp.stack(inv2s)
    g2s = jnp.stack([params['sub%d' % i]['g2'][None] for i in range(4)])
    be2s = jnp.stack([params['sub%d' % i]['be2'][None] for i in range(4)])
    w3s = jnp.stack([params['sub%d' % i]['W3'] for i in range(4)]
                    ).astype(jnp.bfloat16)
    bias_sum = sum(params['sub%d' % i]['b3'] for i in range(4))[None]
    s = _passc(bs, m2s, inv2s, g2s, be2s, w3s, bias_sum)

    h = _gcn(x, edges, params)
    return jnp.concatenate([h, s], axis=1)


# speedup vs baseline: 2.4500x; 1.2100x over previous
import functools

import jax
import jax.numpy as jnp
from jax import lax
from jax.experimental import pallas as pl
from jax.experimental.pallas import tpu as pltpu
from jax.experimental.pallas import tpu_sc as plsc

N = 10000
D = 128
H1 = 256
H2 = 128
RW = 5
DIN = D + RW

BM_HOP = 200
BM_MLP = 1000


def _hop_mm_kernel(hop_ref, y_ref, z_ref, s1_ref, s2_ref):
    h = hop_ref[...].astype(jnp.bfloat16)
    z = jax.lax.dot_general(h, y_ref[...], (((1,), (0,)), ((), ())),
                            preferred_element_type=jnp.float32)
    z_ref[...] = z

    @pl.when(pl.program_id(0) == 0)
    def _():
        s1_ref[...] = jnp.zeros_like(s1_ref)
        s2_ref[...] = jnp.zeros_like(s2_ref)

    s1_ref[...] += jnp.sum(z, axis=0, keepdims=True)
    s2_ref[...] += jnp.sum(z * z, axis=0, keepdims=True)


def _hop_mm(hop, y_bf16):
    return pl.pallas_call(
        _hop_mm_kernel,
        grid=(N // BM_HOP,),
        in_specs=[pl.BlockSpec((BM_HOP, N), lambda i: (i, 0)),
                  pl.BlockSpec((N, H1), lambda i: (0, 0))],
        out_specs=[pl.BlockSpec((BM_HOP, H1), lambda i: (i, 0)),
                   pl.BlockSpec((1, H1), lambda i: (0, 0)),
                   pl.BlockSpec((1, H1), lambda i: (0, 0))],
        out_shape=[jax.ShapeDtypeStruct((N, H1), jnp.float32),
                   jax.ShapeDtypeStruct((1, H1), jnp.float32),
                   jax.ShapeDtypeStruct((1, H1), jnp.float32)],
    )(hop, y_bf16)


def _yprep_kernel(xc_ref, w_ref, y_ref, s1_ref, s2_ref):
    y = jax.lax.dot_general(xc_ref[...], w_ref[0], (((1,), (0,)), ((), ())),
                            preferred_element_type=jnp.float32)
    y_ref[0] = y.astype(jnp.bfloat16)

    @pl.when(pl.program_id(1) == 0)
    def _():
        s1_ref[...] = jnp.zeros_like(s1_ref)
        s2_ref[...] = jnp.zeros_like(s2_ref)

    s1_ref[...] += jnp.sum(y, axis=0, keepdims=True)[None]
    s2_ref[...] += jnp.sum(y * y, axis=0, keepdims=True)[None]


def _yprep(xc_bf16, w1s_bf16):
    return pl.pallas_call(
        _yprep_kernel,
        grid=(4, N // BM_MLP),
        in_specs=[pl.BlockSpec((BM_MLP, DIN), lambda b, i: (i, 0)),
                  pl.BlockSpec((1, DIN, H1), lambda b, i: (b, 0, 0))],
        out_specs=[pl.BlockSpec((1, BM_MLP, H1), lambda b, i: (b, i, 0)),
                   pl.BlockSpec((1, 1, H1), lambda b, i: (b, 0, 0)),
                   pl.BlockSpec((1, 1, H1), lambda b, i: (b, 0, 0))],
        out_shape=[jax.ShapeDtypeStruct((4, N, H1), jnp.bfloat16),
                   jax.ShapeDtypeStruct((4, 1, H1), jnp.float32),
                   jax.ShapeDtypeStruct((4, 1, H1), jnp.float32)],
    )(xc_bf16, w1s_bf16)


def _passb_kernel(a_ref, m_ref, inv_ref, g_ref, be_ref, w2_ref,
                  b_ref, s1_ref, s2_ref):
    a = a_ref[...].astype(jnp.float32)
    h = (a - m_ref[...]) * inv_ref[...] * g_ref[...] + be_ref[...]
    h = jnp.maximum(h, 0.0).astype(jnp.bfloat16)
    b = jax.lax.dot_general(h, w2_ref[...], (((1,), (0,)), ((), ())),
                            preferred_element_type=jnp.float32)
    b_ref[...] = b

    @pl.when(pl.program_id(0) == 0)
    def _():
        s1_ref[...] = jnp.zeros_like(s1_ref)
        s2_ref[...] = jnp.zeros_like(s2_ref)

    s1_ref[...] += jnp.sum(b, axis=0, keepdims=True)
    s2_ref[...] += jnp.sum(b * b, axis=0, keepdims=True)


def _passb(a, m1, inv1, g1, be1, w2_bf16):
    return pl.pallas_call(
        _passb_kernel,
        grid=(N // BM_MLP,),
        in_specs=[pl.BlockSpec((BM_MLP, H1), lambda i: (i, 0)),
                  pl.BlockSpec((1, H1), lambda i: (0, 0)),
                  pl.BlockSpec((1, H1), lambda i: (0, 0)),
                  pl.BlockSpec((1, H1), lambda i: (0, 0)),
                  pl.BlockSpec((1, H1), lambda i: (0, 0)),
                  pl.BlockSpec((H1, H2), lambda i: (0, 0))],
        out_specs=[pl.BlockSpec((BM_MLP, H2), lambda i: (i, 0)),
                   pl.BlockSpec((1, H2), lambda i: (0, 0)),
                   pl.BlockSpec((1, H2), lambda i: (0, 0))],
        out_shape=[jax.ShapeDtypeStruct((N, H2), jnp.float32),
                   jax.ShapeDtypeStruct((1, H2), jnp.float32),
                   jax.ShapeDtypeStruct((1, H2), jnp.float32)],
    )(a, m1, inv1, g1, be1, w2_bf16)


def _passc_kernel(b0, b1, b2, b3, m_ref, inv_ref, g_ref, be_ref, w3_ref,
                  bias_ref, s_ref):
    acc = bias_ref[...] * jnp.ones((b0.shape[0], 1), jnp.float32)
    for i, bref in enumerate((b0, b1, b2, b3)):
        h = (bref[...] - m_ref[i]) * inv_ref[i] * g_ref[i] + be_ref[i]
        h = jnp.maximum(h, 0.0).astype(jnp.bfloat16)
        acc += jax.lax.dot_general(h, w3_ref[i], (((1,), (0,)), ((), ())),
                                   preferred_element_type=jnp.float32)
    s_ref[...] = acc


def _passc(bs, m2s, inv2s, g2s, be2s, w3s_bf16, bias_sum):
    vec = lambda: pl.BlockSpec((4, 1, H2), lambda i: (0, 0, 0))
    return pl.pallas_call(
        _passc_kernel,
        grid=(N // BM_MLP,),
        in_specs=[pl.BlockSpec((BM_MLP, H2), lambda i: (i, 0))] * 4
                 + [vec(), vec(), vec(), vec(),
                    pl.BlockSpec((4, H2, H2), lambda i: (0, 0, 0)),
                    pl.BlockSpec((1, H2), lambda i: (0, 0))],
        out_specs=pl.BlockSpec((BM_MLP, H2), lambda i: (i, 0)),
        out_shape=jax.ShapeDtypeStruct((N, H2), jnp.float32),
    )(*bs, m2s, inv2s, g2s, be2s, w3s_bf16, bias_sum)


def _finalize_stats(s1, s2):
    m = s1 / N
    v = jnp.maximum(s2 / N - m * m, 0.0)
    return m, jax.lax.rsqrt(v + 1e-5)


N_PAD = 10240
STRIPE = N_PAD // 16
K_E = 128
E_TOT = 320000 + N
PER_TILE = 10752
NB = 1
Q = PER_TILE // NB
CQ = Q // K_E
E_PAD = 32 * PER_TILE
K_DEG = 256
_SC_MESH = dict(core_axis_name="c", subcore_axis_name="s")


def _deg_sc(dst_pad):
    @functools.partial(
        pl.kernel,
        out_type=jax.ShapeDtypeStruct((2, N_PAD, 16), jnp.float32),
        mesh=plsc.VectorSubcoreMesh(**_SC_MESH),
        compiler_params=pltpu.CompilerParams(use_tc_tiling_on_sc=False),
        scratch_types=[
            pltpu.VMEM((K_DEG,), jnp.int32),
            pltpu.VMEM((K_DEG, 16), jnp.float32),
            pltpu.VMEM_SHARED((N_PAD, 16), jnp.float32),
            pltpu.VMEM((STRIPE, 16), jnp.float32),
        ],
    )
    def k(dst_hbm, out, dbuf, ones_v, acc, zbuf):
        cid = lax.axis_index("c")
        sid = lax.axis_index("s")
        wid = cid * 16 + sid
        def fill(i, carry):
            ones_v[i, pl.ds(0, 16)] = jnp.ones((16,), jnp.float32)
            return carry
        lax.fori_loop(0, K_DEG, fill, 0)
        def zf(i, carry):
            zbuf[i, pl.ds(0, 16)] = jnp.zeros((16,), jnp.float32)
            return carry
        lax.fori_loop(0, STRIPE, zf, 0)
        pltpu.sync_copy(zbuf, acc.at[pl.ds(sid * STRIPE, STRIPE)])
        plsc.subcore_barrier()

        def chunk(c, carry):
            pltpu.sync_copy(
                dst_hbm.at[pl.ds(wid * PER_TILE + c * K_DEG, K_DEG)], dbuf)
            pltpu.sync_copy(ones_v, acc.at[dbuf], add=True)
            return carry
        lax.fori_loop(0, PER_TILE // K_DEG, chunk, 0)
        plsc.subcore_barrier()
        pltpu.sync_copy(acc.at[pl.ds(sid * STRIPE, STRIPE)],
                        out.at[cid, pl.ds(sid * STRIPE, STRIPE)])

    return k(dst_pad)


def _gcn_agg_sc(src_pad, dst_pad, tabs, zer):
    nt = len(tabs)

    @functools.partial(
        pl.kernel,
        out_type=jax.ShapeDtypeStruct((2, nt, N_PAD, 128), jnp.bfloat16),
        mesh=plsc.VectorSubcoreMesh(**_SC_MESH),
        compiler_params=pltpu.CompilerParams(use_tc_tiling_on_sc=False),
        scratch_types=[
            pltpu.VMEM((Q,), jnp.int32),
            pltpu.VMEM((Q,), jnp.int32),
            pltpu.VMEM((2 * K_E, 128), jnp.bfloat16),
            pltpu.VMEM_SHARED((N_PAD, 128), jnp.bfloat16),
            pltpu.VMEM_SHARED((N_PAD, 128), jnp.bfloat16),
            pltpu.SemaphoreType.DMA,
            pltpu.SemaphoreType.DMA,
            pltpu.SemaphoreType.DMA,
            pltpu.SemaphoreType.DMA,
        ],
    )
    def k(src_hbm, dst_hbm, zer_hbm, *rest):
        tab_refs = rest[:nt]
        out = rest[nt]
        sidx, didx, rows, acc, tsp, sem0, sem1, sem2, sem3 = rest[nt + 1:]
        sems = (sem0, sem1)
        ssems = (sem2, sem3)
        cid = lax.axis_index("c")
        sid = lax.axis_index("s")
        wid = cid * 16 + sid
        ebase = wid * PER_TILE

        for t in range(nt):
            pltpu.sync_copy(tab_refs[t].at[pl.ds(sid * STRIPE, STRIPE)],
                            tsp.at[pl.ds(sid * STRIPE, STRIPE)])
            pltpu.sync_copy(zer_hbm, acc.at[pl.ds(sid * STRIPE, STRIPE)])
            plsc.subcore_barrier()
            tab = tsp

            for n in range(NB):
                qbase = ebase + n * Q
                pltpu.sync_copy(src_hbm.at[pl.ds(qbase, Q)], sidx)
                pltpu.sync_copy(dst_hbm.at[pl.ds(qbase, Q)], didx)
                pltpu.async_copy(tab.at[sidx.at[pl.ds(0, K_E)]],
                                 rows.at[pl.ds(0, K_E)], sems[0])

                def pair(j, carry):
                    for b in (0, 1):
                        c = 2 * j + b
                        nxt = 1 - b
                        pltpu.make_async_copy(
                            tab.at[sidx.at[pl.ds(c * K_E, K_E)]],
                            rows.at[pl.ds(b * K_E, K_E)], sems[b]).wait()
                        pltpu.async_copy(rows.at[pl.ds(b * K_E, K_E)],
                                         acc.at[didx.at[pl.ds(c * K_E, K_E)]],
                                         ssems[b], add=True)
                        if b == 0:
                            @pl.when(j >= 1)
                            def _():
                                pltpu.make_async_copy(
                                    rows.at[pl.ds(K_E, K_E)],
                                    acc.at[didx.at[pl.ds((c - 1) * K_E, K_E)]],
                                    ssems[1]).wait()
                            pltpu.async_copy(
                                tab.at[sidx.at[pl.ds((c + 1) * K_E, K_E)]],
                                rows.at[pl.ds(K_E, K_E)], sems[1])
                        else:
                            pltpu.make_async_copy(
                                rows.at[pl.ds(0, K_E)],
                                acc.at[didx.at[pl.ds((c - 1) * K_E, K_E)]],
                                ssems[0]).wait()
                            @pl.when(c + 1 < CQ)
                            def _():
                                pltpu.async_copy(
                                    tab.at[sidx.at[pl.ds((c + 1) * K_E, K_E)]],
                                    rows.at[pl.ds(0, K_E)], sems[0])
                    return carry
                lax.fori_loop(0, CQ // 2, pair, 0)
                pltpu.make_async_copy(
                    rows.at[pl.ds(K_E, K_E)],
                    acc.at[didx.at[pl.ds((CQ - 1) * K_E, K_E)]],
                    ssems[1]).wait()

            plsc.subcore_barrier()
            pltpu.sync_copy(acc.at[pl.ds(sid * STRIPE, STRIPE)],
                            out.at[cid, t, pl.ds(sid * STRIPE, STRIPE)])
            plsc.subcore_barrier()

    return k(src_pad, dst_pad, zer, *tabs)


def _dis_block(degp):
    deg = degp[0, :, :1] + degp[1, :, :1]
    return jax.lax.rsqrt(jnp.maximum(deg, 1e-12))


def _psum(aggp_ref, c):
    return (aggp_ref[0, c].astype(jnp.float32)
            + aggp_ref[1, c].astype(jnp.float32))


def _prep1_kernel(x_ref, degp_ref, o_ref):
    o_ref[...] = (x_ref[...] * _dis_block(degp_ref)).astype(jnp.bfloat16)


def _prep2_kernel(aggp_ref, degp_ref, w1_ref, b1_ref, w2_ref, o_ref):
    dis = _dis_block(degp_ref)
    p = _psum(aggp_ref, 0) * dis
    h = jax.lax.dot_general(p, w1_ref[...], (((1,), (0,)), ((), ())),
                            preferred_element_type=jnp.float32) + b1_ref[...]
    h = jnp.maximum(h, 0.0)
    o_ref[...] = (jax.lax.dot_general(
        h, w2_ref[...], (((1,), (0,)), ((), ())),
        preferred_element_type=jnp.float32) * dis).astype(jnp.bfloat16)


def _prep23_kernel(nc, aggp_ref, degp_ref, w_ref, bias_ref, o_ref):
    dis = _dis_block(degp_ref)
    hs = []
    for c in range(nc):
        p = _psum(aggp_ref, c)
        hs.append(jnp.maximum(p * dis + bias_ref[:, c * 128:(c + 1) * 128],
                              0.0))
    h = jnp.concatenate(hs, axis=1) if nc > 1 else hs[0]
    o_ref[...] = (jax.lax.dot_general(
        h, w_ref[...], (((1,), (0,)), ((), ())),
        preferred_element_type=jnp.float32) * dis).astype(o_ref.dtype)


def _epi_kernel(aggp_ref, degp_ref, bias_ref, o_ref):
    dis = _dis_block(degp_ref)
    o_ref[...] = jnp.maximum(_psum(aggp_ref, 0) * dis + bias_ref[...], 0.0)


_BMG = 2000


def _gcn_tc_specs(nc, hw):
    return [pl.BlockSpec((2, nc, _BMG, 128), lambda i: (0, 0, i, 0)),
            pl.BlockSpec((2, _BMG, 16), lambda i: (0, i, 0)),
            pl.BlockSpec(hw, lambda i: (0, 0)),
            pl.BlockSpec((1, 128 * nc), lambda i: (0, 0))]


def _gcn(x, edges, params, partners):
    loops = jnp.arange(N, dtype=jnp.int32)
    fill_s = jnp.full((E_PAD - E_TOT,), N, jnp.int32)
    fill_d = jnp.full((32 * PER_TILE - E_TOT,), N, jnp.int32)
    src = jnp.concatenate([edges[0], loops, fill_s])
    dst = jnp.concatenate([edges[1], loops, fill_d])

    degp = _deg_sc(dst)
    zer = jnp.zeros((STRIPE, 128), jnp.bfloat16)

    t1 = pl.pallas_call(
        _prep1_kernel,
        grid=(N // _BMG,),
        in_specs=[pl.BlockSpec((_BMG, D), lambda i: (i, 0)),
                  pl.BlockSpec((2, _BMG, 16), lambda i: (0, i, 0))],
        out_specs=pl.BlockSpec((_BMG, 128), lambda i: (i, 0)),
        out_shape=jax.ShapeDtypeStruct((N_PAD, 128), jnp.bfloat16),
    )(x, degp)
    agg1 = _gcn_agg_sc(src, dst, [t1], zer)
    agg1, z1 = lax.optimization_barrier((agg1, partners[0]()))

    xw2 = pl.pallas_call(
        _prep2_kernel,
        grid=(N // _BMG,),
        in_specs=[pl.BlockSpec((2, 1, _BMG, 128), lambda i: (0, 0, i, 0)),
                  pl.BlockSpec((2, _BMG, 16), lambda i: (0, i, 0)),
                  pl.BlockSpec((D, H1), lambda i: (0, 0)),
                  pl.BlockSpec((1, H1), lambda i: (0, 0)),
                  pl.BlockSpec((H1, H2), lambda i: (0, 0))],
        out_specs=pl.BlockSpec((_BMG, 128), lambda i: (i, 0)),
        out_shape=jax.ShapeDtypeStruct((N_PAD, 128), jnp.bfloat16),
    )(agg1, degp, params['gcn1']['W'], params['gcn1']['b'][None],
      params['gcn2']['W'])
    agg2 = _gcn_agg_sc(src, dst, [xw2], zer)
    agg2, z2 = lax.optimization_barrier((agg2, partners[1]()))

    xw3 = pl.pallas_call(
        functools.partial(_prep23_kernel, 1),
        grid=(N // _BMG,),
        in_specs=_gcn_tc_specs(1, (H2, H2)),
        out_specs=pl.BlockSpec((_BMG, 128), lambda i: (i, 0)),
        out_shape=jax.ShapeDtypeStruct((N_PAD, 128), jnp.bfloat16),
    )(agg2, degp, params['gcn3']['W'], params['gcn2']['b'][None])
    agg3 = _gcn_agg_sc(src, dst, [xw3], zer)
    agg3, z3 = lax.optimization_barrier((agg3, partners[2]()))

    h = pl.pallas_call(
        _epi_kernel,
        grid=(N // _BMG,),
        in_specs=_gcn_tc_specs(1, (H2, H2))[:2]
                 + [pl.BlockSpec((1, H2), lambda i: (0, 0))],
        out_specs=pl.BlockSpec((_BMG, H2), lambda i: (i, 0)),
        out_shape=jax.ShapeDtypeStruct((N, H2), jnp.float32),
    )(agg3, degp, params['gcn3']['b'][None])
    return h, (z1, z2, z3)


def kernel(x, edges, walk_feats, hop1, hop2, hop3, params):
    xc = jnp.concatenate([x, walk_feats[:, :RW]], axis=1).astype(jnp.bfloat16)
    w1s = jnp.stack([params['sub%d' % i]['W1'] for i in range(4)]
                    ).astype(jnp.bfloat16)
    ys, ys1, ys2 = _yprep(xc, w1s)

    partners = [functools.partial(_hop_mm, hop, ys[i + 1])
                for i, hop in enumerate((hop1, hop2, hop3))]
    h, (z1, z2, z3) = _gcn(x, edges, params, partners)

    zs, zs1, zs2 = [None] * 4, [ys1[0]] * 4, [ys2[0]] * 4
    zs[0] = ys[0].astype(jnp.float32)
    for i, z in enumerate((z1, z2, z3)):
        zs[i + 1], zs1[i + 1], zs2[i + 1] = z

    bs, bs1, bs2 = [None] * 4, [None] * 4, [None] * 4
    for i in range(4):
        p = params['sub%d' % i]
        m1, inv1 = _finalize_stats(zs1[i], zs2[i])
        bs[i], bs1[i], bs2[i] = _passb(
            zs[i], m1, inv1, p['g1'][None], p['be1'][None],
            p['W2'].astype(jnp.bfloat16))

    m2s, inv2s = [], []
    for i in range(4):
        m2, inv2 = _finalize_stats(bs1[i], bs2[i])
        m2s.append(m2)
        inv2s.append(inv2)
    m2s = jnp.stack(m2s)
    inv2s = jnp.stack(inv2s)
    g2s = jnp.stack([params['sub%d' % i]['g2'][None] for i in range(4)])
    be2s = jnp.stack([params['sub%d' % i]['be2'][None] for i in range(4)])
    w3s = jnp.stack([params['sub%d' % i]['W3'] for i in range(4)]
                    ).astype(jnp.bfloat16)
    bias_sum = sum(params['sub%d' % i]['b3'] for i in range(4))[None]
    s = _passc(bs, m2s, inv2s, g2s, be2s, w3s, bias_sum)

    return jnp.concatenate([h, s], axis=1)


# speedup vs baseline: 2.5270x; 1.0314x over previous
import functools

import jax
import jax.numpy as jnp
from jax import lax
from jax.experimental import pallas as pl
from jax.experimental.pallas import tpu as pltpu
from jax.experimental.pallas import tpu_sc as plsc

N = 10000
D = 128
H1 = 256
H2 = 128
RW = 5
DIN = D + RW

BM_HOP = 200
BM_MLP = 1000


def _hop_mm_kernel(hop_ref, y_ref, z_ref, s1_ref, s2_ref):
    h = hop_ref[...].astype(jnp.bfloat16)
    z = jax.lax.dot_general(h, y_ref[...], (((1,), (0,)), ((), ())),
                            preferred_element_type=jnp.float32)
    z_ref[...] = z

    @pl.when(pl.program_id(0) == 0)
    def _():
        s1_ref[...] = jnp.zeros_like(s1_ref)
        s2_ref[...] = jnp.zeros_like(s2_ref)

    s1_ref[...] += jnp.sum(z, axis=0, keepdims=True)
    s2_ref[...] += jnp.sum(z * z, axis=0, keepdims=True)


def _hop_mm(hop, y_bf16):
    return pl.pallas_call(
        _hop_mm_kernel,
        grid=(N // BM_HOP,),
        in_specs=[pl.BlockSpec((BM_HOP, N), lambda i: (i, 0)),
                  pl.BlockSpec((N, H1), lambda i: (0, 0))],
        out_specs=[pl.BlockSpec((BM_HOP, H1), lambda i: (i, 0)),
                   pl.BlockSpec((1, H1), lambda i: (0, 0)),
                   pl.BlockSpec((1, H1), lambda i: (0, 0))],
        out_shape=[jax.ShapeDtypeStruct((N, H1), jnp.float32),
                   jax.ShapeDtypeStruct((1, H1), jnp.float32),
                   jax.ShapeDtypeStruct((1, H1), jnp.float32)],
    )(hop, y_bf16)


def _yprep_kernel(xc_ref, w_ref, y_ref, s1_ref, s2_ref):
    y = jax.lax.dot_general(xc_ref[...], w_ref[0], (((1,), (0,)), ((), ())),
                            preferred_element_type=jnp.float32)
    y_ref[0] = y.astype(jnp.bfloat16)

    @pl.when(pl.program_id(1) == 0)
    def _():
        s1_ref[...] = jnp.zeros_like(s1_ref)
        s2_ref[...] = jnp.zeros_like(s2_ref)

    s1_ref[...] += jnp.sum(y, axis=0, keepdims=True)[None]
    s2_ref[...] += jnp.sum(y * y, axis=0, keepdims=True)[None]


def _yprep(xc_bf16, w1s_bf16):
    return pl.pallas_call(
        _yprep_kernel,
        grid=(4, N // BM_MLP),
        in_specs=[pl.BlockSpec((BM_MLP, DIN), lambda b, i: (i, 0)),
                  pl.BlockSpec((1, DIN, H1), lambda b, i: (b, 0, 0))],
        out_specs=[pl.BlockSpec((1, BM_MLP, H1), lambda b, i: (b, i, 0)),
                   pl.BlockSpec((1, 1, H1), lambda b, i: (b, 0, 0)),
                   pl.BlockSpec((1, 1, H1), lambda b, i: (b, 0, 0))],
        out_shape=[jax.ShapeDtypeStruct((4, N, H1), jnp.bfloat16),
                   jax.ShapeDtypeStruct((4, 1, H1), jnp.float32),
                   jax.ShapeDtypeStruct((4, 1, H1), jnp.float32)],
    )(xc_bf16, w1s_bf16)


def _passb_kernel(a_ref, m_ref, inv_ref, g_ref, be_ref, w2_ref,
                  b_ref, s1_ref, s2_ref):
    a = a_ref[...].astype(jnp.float32)
    h = (a - m_ref[...]) * inv_ref[...] * g_ref[...] + be_ref[...]
    h = jnp.maximum(h, 0.0).astype(jnp.bfloat16)
    b = jax.lax.dot_general(h, w2_ref[...], (((1,), (0,)), ((), ())),
                            preferred_element_type=jnp.float32)
    b_ref[...] = b

    @pl.when(pl.program_id(0) == 0)
    def _():
        s1_ref[...] = jnp.zeros_like(s1_ref)
        s2_ref[...] = jnp.zeros_like(s2_ref)

    s1_ref[...] += jnp.sum(b, axis=0, keepdims=True)
    s2_ref[...] += jnp.sum(b * b, axis=0, keepdims=True)


def _passb(a, m1, inv1, g1, be1, w2_bf16):
    return pl.pallas_call(
        _passb_kernel,
        grid=(N // BM_MLP,),
        in_specs=[pl.BlockSpec((BM_MLP, H1), lambda i: (i, 0)),
                  pl.BlockSpec((1, H1), lambda i: (0, 0)),
                  pl.BlockSpec((1, H1), lambda i: (0, 0)),
                  pl.BlockSpec((1, H1), lambda i: (0, 0)),
                  pl.BlockSpec((1, H1), lambda i: (0, 0)),
                  pl.BlockSpec((H1, H2), lambda i: (0, 0))],
        out_specs=[pl.BlockSpec((BM_MLP, H2), lambda i: (i, 0)),
                   pl.BlockSpec((1, H2), lambda i: (0, 0)),
                   pl.BlockSpec((1, H2), lambda i: (0, 0))],
        out_shape=[jax.ShapeDtypeStruct((N, H2), jnp.float32),
                   jax.ShapeDtypeStruct((1, H2), jnp.float32),
                   jax.ShapeDtypeStruct((1, H2), jnp.float32)],
    )(a, m1, inv1, g1, be1, w2_bf16)


def _passc_kernel(b0, b1, b2, b3, m_ref, inv_ref, g_ref, be_ref, w3_ref,
                  bias_ref, aggp_ref, degp_ref, b3g_ref, o_ref):
    dis = _dis_block(degp_ref)
    o_ref[:, :H2] = jnp.maximum(_psum(aggp_ref, 0) * dis + b3g_ref[...], 0.0)
    acc = bias_ref[...] * jnp.ones((b0.shape[0], 1), jnp.float32)
    for i, bref in enumerate((b0, b1, b2, b3)):
        h = (bref[...] - m_ref[i]) * inv_ref[i] * g_ref[i] + be_ref[i]
        h = jnp.maximum(h, 0.0).astype(jnp.bfloat16)
        acc += jax.lax.dot_general(h, w3_ref[i], (((1,), (0,)), ((), ())),
                                   preferred_element_type=jnp.float32)
    o_ref[:, H2:] = acc


def _passc(bs, m2s, inv2s, g2s, be2s, w3s_bf16, bias_sum, agg3, degp, b3g):
    vec = lambda: pl.BlockSpec((4, 1, H2), lambda i: (0, 0, 0))
    return pl.pallas_call(
        _passc_kernel,
        grid=(N // BM_MLP,),
        in_specs=[pl.BlockSpec((BM_MLP, H2), lambda i: (i, 0))] * 4
                 + [vec(), vec(), vec(), vec(),
                    pl.BlockSpec((4, H2, H2), lambda i: (0, 0, 0)),
                    pl.BlockSpec((1, H2), lambda i: (0, 0)),
                    pl.BlockSpec((2, 1, BM_MLP, 128), lambda i: (0, 0, i, 0)),
                    pl.BlockSpec((2, BM_MLP, 16), lambda i: (0, i, 0)),
                    pl.BlockSpec((1, H2), lambda i: (0, 0))],
        out_specs=pl.BlockSpec((BM_MLP, 2 * H2), lambda i: (i, 0)),
        out_shape=jax.ShapeDtypeStruct((N, 2 * H2), jnp.float32),
    )(*bs, m2s, inv2s, g2s, be2s, w3s_bf16, bias_sum, agg3, degp, b3g)


def _finalize_stats(s1, s2):
    m = s1 / N
    v = jnp.maximum(s2 / N - m * m, 0.0)
    return m, jax.lax.rsqrt(v + 1e-5)


N_PAD = 10240
STRIPE = N_PAD // 16
K_E = 128
E_TOT = 320000 + N
PER_TILE = 10752
NB = 1
Q = PER_TILE // NB
CQ = Q // K_E
E_PAD = 32 * PER_TILE
K_DEG = 256
_SC_MESH = dict(core_axis_name="c", subcore_axis_name="s")


def _deg_sc(dst_pad):
    @functools.partial(
        pl.kernel,
        out_type=jax.ShapeDtypeStruct((2, N_PAD, 16), jnp.float32),
        mesh=plsc.VectorSubcoreMesh(**_SC_MESH),
        compiler_params=pltpu.CompilerParams(use_tc_tiling_on_sc=False),
        scratch_types=[
            pltpu.VMEM((K_DEG,), jnp.int32),
            pltpu.VMEM((K_DEG, 16), jnp.float32),
            pltpu.VMEM_SHARED((N_PAD, 16), jnp.float32),
            pltpu.VMEM((STRIPE, 16), jnp.float32),
        ],
    )
    def k(dst_hbm, out, dbuf, ones_v, acc, zbuf):
        cid = lax.axis_index("c")
        sid = lax.axis_index("s")
        wid = cid * 16 + sid
        def fill(i, carry):
            ones_v[i, pl.ds(0, 16)] = jnp.ones((16,), jnp.float32)
            return carry
        lax.fori_loop(0, K_DEG, fill, 0)
        def zf(i, carry):
            zbuf[i, pl.ds(0, 16)] = jnp.zeros((16,), jnp.float32)
            return carry
        lax.fori_loop(0, STRIPE, zf, 0)
        pltpu.sync_copy(zbuf, acc.at[pl.ds(sid * STRIPE, STRIPE)])
        plsc.subcore_barrier()

        def chunk(c, carry):
            pltpu.sync_copy(
                dst_hbm.at[pl.ds(wid * PER_TILE + c * K_DEG, K_DEG)], dbuf)
            pltpu.sync_copy(ones_v, acc.at[dbuf], add=True)
            return carry
        lax.fori_loop(0, PER_TILE // K_DEG, chunk, 0)
        plsc.subcore_barrier()
        pltpu.sync_copy(acc.at[pl.ds(sid * STRIPE, STRIPE)],
                        out.at[cid, pl.ds(sid * STRIPE, STRIPE)])

    return k(dst_pad)


def _gcn_agg_sc(src_pad, dst_pad, tabs, zer):
    nt = len(tabs)

    @functools.partial(
        pl.kernel,
        out_type=jax.ShapeDtypeStruct((2, nt, N_PAD, 128), jnp.bfloat16),
        mesh=plsc.VectorSubcoreMesh(**_SC_MESH),
        compiler_params=pltpu.CompilerParams(use_tc_tiling_on_sc=False),
        scratch_types=[
            pltpu.VMEM((Q,), jnp.int32),
            pltpu.VMEM((Q,), jnp.int32),
            pltpu.VMEM((2 * K_E, 128), jnp.bfloat16),
            pltpu.VMEM_SHARED((N_PAD, 128), jnp.bfloat16),
            pltpu.VMEM_SHARED((N_PAD, 128), jnp.bfloat16),
            pltpu.SemaphoreType.DMA,
            pltpu.SemaphoreType.DMA,
            pltpu.SemaphoreType.DMA,
            pltpu.SemaphoreType.DMA,
        ],
    )
    def k(src_hbm, dst_hbm, zer_hbm, *rest):
        tab_refs = rest[:nt]
        out = rest[nt]
        sidx, didx, rows, acc, tsp, sem0, sem1, sem2, sem3 = rest[nt + 1:]
        sems = (sem0, sem1)
        ssems = (sem2, sem3)
        cid = lax.axis_index("c")
        sid = lax.axis_index("s")
        wid = cid * 16 + sid
        ebase = wid * PER_TILE

        for t in range(nt):
            pltpu.sync_copy(tab_refs[t].at[pl.ds(sid * STRIPE, STRIPE)],
                            tsp.at[pl.ds(sid * STRIPE, STRIPE)])
            pltpu.sync_copy(zer_hbm, acc.at[pl.ds(sid * STRIPE, STRIPE)])
            plsc.subcore_barrier()
            tab = tsp

            for n in range(NB):
                qbase = ebase + n * Q
                pltpu.sync_copy(src_hbm.at[pl.ds(qbase, Q)], sidx)
                pltpu.sync_copy(dst_hbm.at[pl.ds(qbase, Q)], didx)
                pltpu.async_copy(tab.at[sidx.at[pl.ds(0, K_E)]],
                                 rows.at[pl.ds(0, K_E)], sems[0])

                def pair(j, carry):
                    for b in (0, 1):
                        c = 2 * j + b
                        nxt = 1 - b
                        pltpu.make_async_copy(
                            tab.at[sidx.at[pl.ds(c * K_E, K_E)]],
                            rows.at[pl.ds(b * K_E, K_E)], sems[b]).wait()
                        pltpu.async_copy(rows.at[pl.ds(b * K_E, K_E)],
                                         acc.at[didx.at[pl.ds(c * K_E, K_E)]],
                                         ssems[b], add=True)
                        if b == 0:
                            @pl.when(j >= 1)
                            def _():
                                pltpu.make_async_copy(
                                    rows.at[pl.ds(K_E, K_E)],
                                    acc.at[didx.at[pl.ds((c - 1) * K_E, K_E)]],
                                    ssems[1]).wait()
                            pltpu.async_copy(
                                tab.at[sidx.at[pl.ds((c + 1) * K_E, K_E)]],
                                rows.at[pl.ds(K_E, K_E)], sems[1])
                        else:
                            pltpu.make_async_copy(
                                rows.at[pl.ds(0, K_E)],
                                acc.at[didx.at[pl.ds((c - 1) * K_E, K_E)]],
                                ssems[0]).wait()
                            @pl.when(c + 1 < CQ)
                            def _():
                                pltpu.async_copy(
                                    tab.at[sidx.at[pl.ds((c + 1) * K_E, K_E)]],
                                    rows.at[pl.ds(0, K_E)], sems[0])
                    return carry
                lax.fori_loop(0, CQ // 2, pair, 0)
                pltpu.make_async_copy(
                    rows.at[pl.ds(K_E, K_E)],
                    acc.at[didx.at[pl.ds((CQ - 1) * K_E, K_E)]],
                    ssems[1]).wait()

            plsc.subcore_barrier()
            pltpu.sync_copy(acc.at[pl.ds(sid * STRIPE, STRIPE)],
                            out.at[cid, t, pl.ds(sid * STRIPE, STRIPE)])
            plsc.subcore_barrier()

    return k(src_pad, dst_pad, zer, *tabs)


def _dis_block(degp):
    deg = degp[0, :, :1] + degp[1, :, :1]
    return jax.lax.rsqrt(jnp.maximum(deg, 1e-12))


def _psum(aggp_ref, c):
    return (aggp_ref[0, c].astype(jnp.float32)
            + aggp_ref[1, c].astype(jnp.float32))


def _prep1_kernel(x_ref, degp_ref, o_ref):
    o_ref[...] = (x_ref[...] * _dis_block(degp_ref)).astype(jnp.bfloat16)


def _prep2_kernel(aggp_ref, degp_ref, w1_ref, b1_ref, w2_ref, o_ref):
    dis = _dis_block(degp_ref)
    p = _psum(aggp_ref, 0) * dis
    h = jax.lax.dot_general(p, w1_ref[...], (((1,), (0,)), ((), ())),
                            preferred_element_type=jnp.float32) + b1_ref[...]
    h = jnp.maximum(h, 0.0)
    o_ref[...] = (jax.lax.dot_general(
        h, w2_ref[...], (((1,), (0,)), ((), ())),
        preferred_element_type=jnp.float32) * dis).astype(jnp.bfloat16)


def _prep23_kernel(nc, aggp_ref, degp_ref, w_ref, bias_ref, o_ref):
    dis = _dis_block(degp_ref)
    hs = []
    for c in range(nc):
        p = _psum(aggp_ref, c)
        hs.append(jnp.maximum(p * dis + bias_ref[:, c * 128:(c + 1) * 128],
                              0.0))
    h = jnp.concatenate(hs, axis=1) if nc > 1 else hs[0]
    o_ref[...] = (jax.lax.dot_general(
        h, w_ref[...], (((1,), (0,)), ((), ())),
        preferred_element_type=jnp.float32) * dis).astype(o_ref.dtype)


_BMG = 2000


def _gcn_tc_specs(nc, hw):
    return [pl.BlockSpec((2, nc, _BMG, 128), lambda i: (0, 0, i, 0)),
            pl.BlockSpec((2, _BMG, 16), lambda i: (0, i, 0)),
            pl.BlockSpec(hw, lambda i: (0, 0)),
            pl.BlockSpec((1, 128 * nc), lambda i: (0, 0))]


def _gcn(x, src, dst, degp, params, partners):
    zer = jnp.zeros((STRIPE, 128), jnp.bfloat16)

    t1 = pl.pallas_call(
        _prep1_kernel,
        grid=(N // _BMG,),
        in_specs=[pl.BlockSpec((_BMG, D), lambda i: (i, 0)),
                  pl.BlockSpec((2, _BMG, 16), lambda i: (0, i, 0))],
        out_specs=pl.BlockSpec((_BMG, 128), lambda i: (i, 0)),
        out_shape=jax.ShapeDtypeStruct((N_PAD, 128), jnp.bfloat16),
    )(x, degp)
    agg1 = _gcn_agg_sc(src, dst, [t1], zer)
    agg1, z1 = lax.optimization_barrier((agg1, partners[0]()))

    xw2 = pl.pallas_call(
        _prep2_kernel,
        grid=(N // _BMG,),
        in_specs=[pl.BlockSpec((2, 1, _BMG, 128), lambda i: (0, 0, i, 0)),
                  pl.BlockSpec((2, _BMG, 16), lambda i: (0, i, 0)),
                  pl.BlockSpec((D, H1), lambda i: (0, 0)),
                  pl.BlockSpec((1, H1), lambda i: (0, 0)),
                  pl.BlockSpec((H1, H2), lambda i: (0, 0))],
        out_specs=pl.BlockSpec((_BMG, 128), lambda i: (i, 0)),
        out_shape=jax.ShapeDtypeStruct((N_PAD, 128), jnp.bfloat16),
    )(agg1, degp, params['gcn1']['W'], params['gcn1']['b'][None],
      params['gcn2']['W'])
    agg2 = _gcn_agg_sc(src, dst, [xw2], zer)
    agg2, z2 = lax.optimization_barrier((agg2, partners[1]()))

    xw3 = pl.pallas_call(
        functools.partial(_prep23_kernel, 1),
        grid=(N // _BMG,),
        in_specs=_gcn_tc_specs(1, (H2, H2)),
        out_specs=pl.BlockSpec((_BMG, 128), lambda i: (i, 0)),
        out_shape=jax.ShapeDtypeStruct((N_PAD, 128), jnp.bfloat16),
    )(agg2, degp, params['gcn3']['W'], params['gcn2']['b'][None])
    agg3 = _gcn_agg_sc(src, dst, [xw3], zer)
    agg3, z3 = lax.optimization_barrier((agg3, partners[2]()))
    return agg3, (z1, z2, z3)


def kernel(x, edges, walk_feats, hop1, hop2, hop3, params):
    xc = jnp.concatenate([x, walk_feats[:, :RW]], axis=1).astype(jnp.bfloat16)
    w1s = jnp.stack([params['sub%d' % i]['W1'] for i in range(4)]
                    ).astype(jnp.bfloat16)

    loops = jnp.arange(N, dtype=jnp.int32)
    fill_s = jnp.full((E_PAD - E_TOT,), N, jnp.int32)
    fill_d = jnp.full((32 * PER_TILE - E_TOT,), N, jnp.int32)
    src = jnp.concatenate([edges[0], loops, fill_s])
    dst = jnp.concatenate([edges[1], loops, fill_d])

    degp = _deg_sc(dst)
    degp, (ys, ys1, ys2) = lax.optimization_barrier(
        (degp, tuple(_yprep(xc, w1s))))

    partners = [functools.partial(_hop_mm, hop, ys[i + 1])
                for i, hop in enumerate((hop1, hop2, hop3))]
    agg3, (z1, z2, z3) = _gcn(x, src, dst, degp, params, partners)

    zs, zs1, zs2 = [None] * 4, [ys1[0]] * 4, [ys2[0]] * 4
    zs[0] = ys[0].astype(jnp.float32)
    for i, z in enumerate((z1, z2, z3)):
        zs[i + 1], zs1[i + 1], zs2[i + 1] = z

    bs, bs1, bs2 = [None] * 4, [None] * 4, [None] * 4
    for i in range(4):
        p = params['sub%d' % i]
        m1, inv1 = _finalize_stats(zs1[i], zs2[i])
        bs[i], bs1[i], bs2[i] = _passb(
            zs[i], m1, inv1, p['g1'][None], p['be1'][None],
            p['W2'].astype(jnp.bfloat16))

    m2s, inv2s = [], []
    for i in range(4):
        m2, inv2 = _finalize_stats(bs1[i], bs2[i])
        m2s.append(m2)
        inv2s.append(inv2)
    m2s = jnp.stack(m2s)
    inv2s = jnp.stack(inv2s)
    g2s = jnp.stack([params['sub%d' % i]['g2'][None] for i in range(4)])
    be2s = jnp.stack([params['sub%d' % i]['be2'][None] for i in range(4)])
    w3s = jnp.stack([params['sub%d' % i]['W3'] for i in range(4)]
                    ).astype(jnp.bfloat16)
    bias_sum = sum(params['sub%d' % i]['b3'] for i in range(4))[None]
    return _passc(bs, m2s, inv2s, g2s, be2s, w3s, bias_sum,
                  agg3, degp, params['gcn3']['b'][None])
